# baseline (speedup 1.0000x reference)
import functools

import jax
import jax.numpy as jnp
from jax import lax
from jax.experimental import pallas as pl
from jax.experimental.pallas import tpu as pltpu
from jax.experimental.pallas import tpu_sc as plsc

D = 1024
E = 16
K = 2
DFF = 2048
T = 4096
BLK = 256
NB = T * K // BLK + E
P = NB * BLK
FFC = 1024
NFF = DFF // FFC


def _route_body(x_ref, rw_ref, pos_ref, wrep_ref, be_ref, xbf_ref):
    x = x_ref[...]
    rw = rw_ref[...]
    logits = lax.dot_general(x, rw, (((1,), (1,)), ((), ())),
                             preferred_element_type=jnp.float32)
    m = jnp.max(logits, axis=1, keepdims=True)
    p = jnp.exp(logits - m)
    s = p / jnp.sum(p, axis=1, keepdims=True)

    ecols = lax.broadcasted_iota(jnp.int32, (T, E), 1)
    m1 = jnp.max(s, axis=1, keepdims=True)
    a1 = jnp.min(jnp.where(s == m1, ecols, E), axis=1, keepdims=True)
    s_rest = jnp.where(ecols == a1, -jnp.inf, s)
    m2 = jnp.max(s_rest, axis=1, keepdims=True)
    a2 = jnp.min(jnp.where(s_rest == m2, ecols, E), axis=1, keepdims=True)
    denom = m1 + m2 + jnp.finfo(jnp.float32).eps
    w1 = m1 / denom
    w2 = m2 / denom

    oh1 = (ecols == a1).astype(jnp.float32)
    oh2 = (ecols == a2).astype(jnp.float32)

    R = 512
    G = 2 * T // R
    tri = (lax.broadcasted_iota(jnp.int32, (R, R), 0) >
           lax.broadcasted_iota(jnp.int32, (R, R), 1)).astype(jnp.float32)
    within = []
    gsums = []
    for g in range(G):
        oh_g = lax.slice(oh1 if g < G // 2 else oh2,
                         ((g % (G // 2)) * R, 0), ((g % (G // 2)) * R + R, E))
        within.append(lax.dot_general(tri, oh_g, (((1,), (0,)), ((), ())),
                                      preferred_element_type=jnp.float32))
        gsums.append(jnp.sum(oh_g, axis=0, keepdims=True))
    gs = jnp.concatenate(gsums, axis=0)
    trig = (lax.broadcasted_iota(jnp.int32, (G, G), 0) >
            lax.broadcasted_iota(jnp.int32, (G, G), 1)).astype(jnp.float32)
    gpre = lax.dot_general(trig, gs, (((1,), (0,)), ((), ())),
                           preferred_element_type=jnp.float32)

    counts = jnp.sum(gs, axis=0, keepdims=True)
    padded = ((counts.astype(jnp.int32) + (BLK - 1)) // BLK) * BLK
    trie = (lax.broadcasted_iota(jnp.int32, (E, E), 0) >
            lax.broadcasted_iota(jnp.int32, (E, E), 1)).astype(jnp.float32)
    starts = lax.dot_general(padded.astype(jnp.float32), trie,
                             (((1,), (1,)), ((), ())),
                             preferred_element_type=jnp.float32)
    starts_i = starts.astype(jnp.int32)

    for g in range(G):
        oh_g = lax.slice(oh1 if g < G // 2 else oh2,
                         ((g % (G // 2)) * R, 0), ((g % (G // 2)) * R + R, E))
        posmat = within[g] + gpre[g:g + 1, :] + starts
        pos_g = jnp.sum(oh_g * posmat, axis=1).astype(jnp.int32)
        krow = 0 if g < G // 2 else 1
        tbase = (g % (G // 2)) * R
        pos_ref[krow, pl.ds(tbase, R)] = pos_g

    wrep_ref[0, :, :] = jnp.broadcast_to(w1, (T, 16))
    wrep_ref[1, :, :] = jnp.broadcast_to(w2, (T, 16))
    xbf_ref[...] = x.astype(jnp.bfloat16)

    brows = lax.broadcasted_iota(jnp.int32, (NB, E), 0) * BLK
    ecols_b = lax.broadcasted_iota(jnp.int32, (NB, E), 1)
    active = (brows >= starts_i) & (brows < starts_i + padded)
    be = jnp.sum(jnp.where(active, ecols_b + 1, 0), axis=1) - 1
    be_ref[0, :] = be
    nb_used = jnp.sum(padded) // BLK
    bcols = lax.broadcasted_iota(jnp.int32, (1, NB), 1)
    be_ref[1, :] = jnp.where(bcols < nb_used, be[None, :], jnp.max(be))[0, :]
    be_ref[2, :] = jnp.minimum(bcols, nb_used - 1)[0, :]


def _route(x2d, router_w, interpret=False):
    return pl.pallas_call(
        _route_body,
        out_shape=(
            jax.ShapeDtypeStruct((K, T), jnp.int32),
            jax.ShapeDtypeStruct((K, T, 16), jnp.float32),
            jax.ShapeDtypeStruct((3, NB), jnp.int32),
            jax.ShapeDtypeStruct((T, D), jnp.bfloat16),
        ),
        interpret=interpret,
    )(x2d, router_w)


def _gemm_body(be_ref, xg_ref, gw_ref, uw_ref, dw_ref, out_ref):
    b = pl.program_id(0)
    e = be_ref[0, b]

    @pl.when(e >= 0)
    def _():
        xb = xg_ref[...].astype(jnp.float32)
        g = lax.dot_general(xb, gw_ref[0], (((1,), (1,)), ((), ())),
                            preferred_element_type=jnp.float32)
        u = lax.dot_general(xb, uw_ref[0], (((1,), (1,)), ((), ())),
                            preferred_element_type=jnp.float32)
        h = jnp.maximum(g, 0.0) * u
        y = lax.dot_general(h, dw_ref[0], (((1,), (1,)), ((), ())),
                            preferred_element_type=jnp.float32)
        out_ref[...] = y.astype(jnp.bfloat16)


def _gemm(xg, gate_w, up_w, down_w, be, interpret=False):
    grid_spec = pltpu.PrefetchScalarGridSpec(
        num_scalar_prefetch=1,
        grid=(NB,),
        in_specs=[
            pl.BlockSpec((BLK, D), lambda b, be: (be[2, b], 0)),
            pl.BlockSpec((1, DFF, D), lambda b, be: (be[1, b], 0, 0)),
            pl.BlockSpec((1, DFF, D), lambda b, be: (be[1, b], 0, 0)),
            pl.BlockSpec((1, D, DFF), lambda b, be: (be[1, b], 0, 0)),
        ],
        out_specs=pl.BlockSpec((BLK, D), lambda b, be: (be[2, b], 0)),
    )
    return pl.pallas_call(
        _gemm_body,
        grid_spec=grid_spec,
        out_shape=jax.ShapeDtypeStruct((P, D), jnp.bfloat16),
        compiler_params=pltpu.CompilerParams(
            dimension_semantics=("arbitrary",)),
        interpret=interpret,
    )(be, xg, gate_w, up_w, down_w)


_NC = 2
_NS = 16
_NW = _NC * _NS
_TPW = T // _NW
_DCH = 64
_CCH = 32

_sc_mesh = plsc.VectorSubcoreMesh(core_axis_name="c", subcore_axis_name="s")


@functools.partial(
    pl.kernel,
    out_type=jax.ShapeDtypeStruct((P, D // 2), jnp.float32),
    mesh=_sc_mesh,
    scratch_types=[
        pltpu.VMEM((_DCH, D // 2), jnp.float32),
        pltpu.VMEM((_DCH,), jnp.int32),
        pltpu.VMEM((_DCH,), jnp.int32),
        pltpu.SemaphoreType.DMA,
        pltpu.SemaphoreType.DMA,
    ],
)
def _dispatch(x_hbm, pos0_hbm, pos1_hbm, xg_hbm, rows_v, i0_v, i1_v, s0, s1):
    wid = lax.axis_index("s") * _NC + lax.axis_index("c")
    for c in range(_TPW // _DCH):
        base = wid * _TPW + c * _DCH
        pltpu.sync_copy(x_hbm.at[pl.ds(base, _DCH)], rows_v)
        pltpu.sync_copy(pos0_hbm.at[pl.ds(base, _DCH)], i0_v)
        pltpu.sync_copy(pos1_hbm.at[pl.ds(base, _DCH)], i1_v)
        cp0 = pltpu.async_copy(rows_v, xg_hbm.at[i0_v], s0)
        cp1 = pltpu.async_copy(rows_v, xg_hbm.at[i1_v], s1)
        cp0.wait()
        cp1.wait()


@functools.partial(
    pl.kernel,
    out_type=(jax.ShapeDtypeStruct((T, D // 2), jnp.float32),
              jax.ShapeDtypeStruct((T, D // 2), jnp.float32)),
    mesh=_sc_mesh,
    scratch_types=[
        pltpu.VMEM((_CCH, D // 2), jnp.float32),
        pltpu.VMEM((_CCH, D // 2), jnp.float32),
        pltpu.VMEM((_CCH,), jnp.int32),
        pltpu.VMEM((_CCH,), jnp.int32),
        pltpu.SemaphoreType.DMA,
        pltpu.SemaphoreType.DMA,
    ],
)
def _gather2(y_hbm, pos0_hbm, pos1_hbm, y0_hbm, y1_hbm,
             a_v, b_v, i0_v, i1_v, s0, s1):
    wid = lax.axis_index("s") * _NC + lax.axis_index("c")
    for c in range(_TPW // _CCH):
        base = wid * _TPW + c * _CCH
        pltpu.sync_copy(pos0_hbm.at[pl.ds(base, _CCH)], i0_v)
        pltpu.sync_copy(pos1_hbm.at[pl.ds(base, _CCH)], i1_v)
        ga = pltpu.async_copy(y_hbm.at[i0_v], a_v, s0)
        gb = pltpu.async_copy(y_hbm.at[i1_v], b_v, s1)
        ga.wait()
        gb.wait()
        pltpu.sync_copy(a_v, y0_hbm.at[pl.ds(base, _CCH)])
        pltpu.sync_copy(b_v, y1_hbm.at[pl.ds(base, _CCH)])


def _finish_body(y0_ref, y1_ref, w_ref, o_ref):
    w0 = w_ref[0, :, 0:1]
    w1 = w_ref[1, :, 0:1]
    o_ref[...] = (y0_ref[...].astype(jnp.float32) * w0
                  + y1_ref[...].astype(jnp.float32) * w1)


def _finish(y0bf, y1bf, wrep):
    tb = T // 8
    return pl.pallas_call(
        _finish_body,
        grid=(8,),
        in_specs=[
            pl.BlockSpec((tb, D), lambda i: (i, 0)),
            pl.BlockSpec((tb, D), lambda i: (i, 0)),
            pl.BlockSpec((K, tb, 16), lambda i: (0, i, 0)),
        ],
        out_specs=pl.BlockSpec((tb, D), lambda i: (i, 0)),
        out_shape=jax.ShapeDtypeStruct((T, D), jnp.float32),
    )(y0bf, y1bf, wrep)


def kernel(x, router_w, gate_w, up_w, down_w):
    B, L, Dd = x.shape
    x2d = x.reshape(T, D)
    pos, wrep, be, xbf = _route(x2d, router_w)

    xp = lax.bitcast_convert_type(
        xbf.reshape(T, D // 2, 2), jnp.float32)
    xg32 = _dispatch(xp, pos[0], pos[1])
    xg = lax.bitcast_convert_type(xg32, jnp.bfloat16).reshape(P, D)
    yg = _gemm(xg, gate_w, up_w, down_w, be)
    yg32 = lax.bitcast_convert_type(
        yg.reshape(P, D // 2, 2), jnp.float32)
    y0p, y1p = _gather2(yg32, pos[0], pos[1])
    y0bf = lax.bitcast_convert_type(y0p, jnp.bfloat16).reshape(T, D)
    y1bf = lax.bitcast_convert_type(y1p, jnp.bfloat16).reshape(T, D)
    return _finish(y0bf, y1bf, wrep).reshape(B, L, Dd)


# speedup vs baseline: 3.4042x; 3.4042x over previous
import functools

import jax
import jax.numpy as jnp
from jax import lax
from jax.experimental import pallas as pl
from jax.experimental.pallas import tpu as pltpu
from jax.experimental.pallas import tpu_sc as plsc

D = 1024
E = 16
K = 2
DFF = 2048
T = 4096
BLK = 256
NB = T * K // BLK + E
P = NB * BLK
FFC = 1024
NFF = DFF // FFC


def _pack_bf16(lo, hi):
    lob = lax.bitcast_convert_type(lo.astype(jnp.bfloat16).astype(jnp.float32),
                                   jnp.int32)
    hib = lax.bitcast_convert_type(hi.astype(jnp.bfloat16).astype(jnp.float32),
                                   jnp.int32)
    word = jnp.bitwise_or(lax.shift_right_logical(lob, 16),
                          jnp.bitwise_and(hib, jnp.int32(-65536)))
    return lax.bitcast_convert_type(word, jnp.float32)


def _unpack_bf16(w):
    wi = lax.bitcast_convert_type(w, jnp.int32)
    lo = lax.bitcast_convert_type(lax.shift_left(wi, 16), jnp.float32)
    hi = lax.bitcast_convert_type(jnp.bitwise_and(wi, jnp.int32(-65536)),
                                  jnp.float32)
    return lo, hi


def _route_body(x_ref, rw_ref, pos_ref, wrep_ref, be_ref, xbf_ref):
    x = x_ref[...]
    rw = rw_ref[...]
    logits = lax.dot_general(x, rw, (((1,), (1,)), ((), ())),
                             preferred_element_type=jnp.float32)
    m = jnp.max(logits, axis=1, keepdims=True)
    p = jnp.exp(logits - m)
    s = p / jnp.sum(p, axis=1, keepdims=True)

    ecols = lax.broadcasted_iota(jnp.int32, (T, E), 1)
    m1 = jnp.max(s, axis=1, keepdims=True)
    a1 = jnp.min(jnp.where(s == m1, ecols, E), axis=1, keepdims=True)
    s_rest = jnp.where(ecols == a1, -jnp.inf, s)
    m2 = jnp.max(s_rest, axis=1, keepdims=True)
    a2 = jnp.min(jnp.where(s_rest == m2, ecols, E), axis=1, keepdims=True)
    denom = m1 + m2 + jnp.finfo(jnp.float32).eps
    w1 = m1 / denom
    w2 = m2 / denom

    oh1 = (ecols == a1).astype(jnp.float32)
    oh2 = (ecols == a2).astype(jnp.float32)

    R = 512
    G = 2 * T // R
    tri = (lax.broadcasted_iota(jnp.int32, (R, R), 0) >
           lax.broadcasted_iota(jnp.int32, (R, R), 1)).astype(jnp.float32)
    within = []
    gsums = []
    for g in range(G):
        oh_g = lax.slice(oh1 if g < G // 2 else oh2,
                         ((g % (G // 2)) * R, 0), ((g % (G // 2)) * R + R, E))
        within.append(lax.dot_general(tri, oh_g, (((1,), (0,)), ((), ())),
                                      preferred_element_type=jnp.float32))
        gsums.append(jnp.sum(oh_g, axis=0, keepdims=True))
    gs = jnp.concatenate(gsums, axis=0)
    trig = (lax.broadcasted_iota(jnp.int32, (G, G), 0) >
            lax.broadcasted_iota(jnp.int32, (G, G), 1)).astype(jnp.float32)
    gpre = lax.dot_general(trig, gs, (((1,), (0,)), ((), ())),
                           preferred_element_type=jnp.float32)

    counts = jnp.sum(gs, axis=0, keepdims=True)
    padded = ((counts.astype(jnp.int32) + (BLK - 1)) // BLK) * BLK
    trie = (lax.broadcasted_iota(jnp.int32, (E, E), 0) >
            lax.broadcasted_iota(jnp.int32, (E, E), 1)).astype(jnp.float32)
    starts = lax.dot_general(padded.astype(jnp.float32), trie,
                             (((1,), (1,)), ((), ())),
                             preferred_element_type=jnp.float32)
    starts_i = starts.astype(jnp.int32)

    for g in range(G):
        oh_g = lax.slice(oh1 if g < G // 2 else oh2,
                         ((g % (G // 2)) * R, 0), ((g % (G // 2)) * R + R, E))
        posmat = within[g] + gpre[g:g + 1, :] + starts
        pos_g = jnp.sum(oh_g * posmat, axis=1).astype(jnp.int32)
        krow = 0 if g < G // 2 else 1
        tbase = (g % (G // 2)) * R
        pos_ref[krow, pl.ds(tbase, R)] = pos_g

    wrep_ref[0, :, :] = jnp.broadcast_to(w1, (T, 16))
    wrep_ref[1, :, :] = jnp.broadcast_to(w2, (T, 16))
    xbf_ref[...] = _pack_bf16(x[:, :D // 2], x[:, D // 2:])

    brows = lax.broadcasted_iota(jnp.int32, (NB, E), 0) * BLK
    ecols_b = lax.broadcasted_iota(jnp.int32, (NB, E), 1)
    active = (brows >= starts_i) & (brows < starts_i + padded)
    be = jnp.sum(jnp.where(active, ecols_b + 1, 0), axis=1) - 1
    be_ref[0, :] = be
    nb_used = jnp.sum(padded) // BLK
    bcols = lax.broadcasted_iota(jnp.int32, (1, NB), 1)
    be_ref[1, :] = jnp.where(bcols < nb_used, be[None, :], jnp.max(be))[0, :]
    be_ref[2, :] = jnp.minimum(bcols, nb_used - 1)[0, :]


def _route(x2d, router_w, interpret=False):
    return pl.pallas_call(
        _route_body,
        out_shape=(
            jax.ShapeDtypeStruct((K, T), jnp.int32),
            jax.ShapeDtypeStruct((K, T, 16), jnp.float32),
            jax.ShapeDtypeStruct((3, NB), jnp.int32),
            jax.ShapeDtypeStruct((T, D // 2), jnp.float32),
        ),
        interpret=interpret,
    )(x2d, router_w)


def _gemm_body(be_ref, xg_ref, gw_ref, uw_ref, dw_ref, out_ref):
    b = pl.program_id(0)
    e = be_ref[0, b]

    @pl.when(e >= 0)
    def _():
        xlo, xhi = _unpack_bf16(xg_ref[...])
        xb = jnp.concatenate([xlo, xhi], axis=1)
        g = lax.dot_general(xb, gw_ref[0], (((1,), (1,)), ((), ())),
                            preferred_element_type=jnp.float32)
        u = lax.dot_general(xb, uw_ref[0], (((1,), (1,)), ((), ())),
                            preferred_element_type=jnp.float32)
        h = jnp.maximum(g, 0.0) * u
        y = lax.dot_general(h, dw_ref[0], (((1,), (1,)), ((), ())),
                            preferred_element_type=jnp.float32)
        out_ref[...] = _pack_bf16(y[:, :D // 2], y[:, D // 2:])


def _gemm(xg, gate_w, up_w, down_w, be, interpret=False):
    grid_spec = pltpu.PrefetchScalarGridSpec(
        num_scalar_prefetch=1,
        grid=(NB,),
        in_specs=[
            pl.BlockSpec((BLK, D // 2), lambda b, be: (be[2, b], 0)),
            pl.BlockSpec((1, DFF, D), lambda b, be: (be[1, b], 0, 0)),
            pl.BlockSpec((1, DFF, D), lambda b, be: (be[1, b], 0, 0)),
            pl.BlockSpec((1, D, DFF), lambda b, be: (be[1, b], 0, 0)),
        ],
        out_specs=pl.BlockSpec((BLK, D // 2), lambda b, be: (be[2, b], 0)),
    )
    return pl.pallas_call(
        _gemm_body,
        grid_spec=grid_spec,
        out_shape=jax.ShapeDtypeStruct((P, D // 2), jnp.float32),
        compiler_params=pltpu.CompilerParams(
            dimension_semantics=("arbitrary",)),
        interpret=interpret,
    )(be, xg, gate_w, up_w, down_w)


_NC = 2
_NS = 16
_NW = _NC * _NS
_TPW = T // _NW
_DCH = 64
_CCH = 32

_sc_mesh = plsc.VectorSubcoreMesh(core_axis_name="c", subcore_axis_name="s")


@functools.partial(
    pl.kernel,
    out_type=jax.ShapeDtypeStruct((P, D // 2), jnp.float32),
    mesh=_sc_mesh,
    scratch_types=[
        pltpu.VMEM((_DCH, D // 2), jnp.float32),
        pltpu.VMEM((_DCH,), jnp.int32),
        pltpu.VMEM((_DCH,), jnp.int32),
        pltpu.SemaphoreType.DMA,
        pltpu.SemaphoreType.DMA,
    ],
)
def _dispatch(x_hbm, pos0_hbm, pos1_hbm, xg_hbm, rows_v, i0_v, i1_v, s0, s1):
    wid = lax.axis_index("s") * _NC + lax.axis_index("c")
    for c in range(_TPW // _DCH):
        base = wid * _TPW + c * _DCH
        pltpu.sync_copy(x_hbm.at[pl.ds(base, _DCH)], rows_v)
        pltpu.sync_copy(pos0_hbm.at[pl.ds(base, _DCH)], i0_v)
        pltpu.sync_copy(pos1_hbm.at[pl.ds(base, _DCH)], i1_v)
        cp0 = pltpu.async_copy(rows_v, xg_hbm.at[i0_v], s0)
        cp1 = pltpu.async_copy(rows_v, xg_hbm.at[i1_v], s1)
        cp0.wait()
        cp1.wait()


@functools.partial(
    pl.kernel,
    out_type=(jax.ShapeDtypeStruct((T, D // 2), jnp.float32),
              jax.ShapeDtypeStruct((T, D // 2), jnp.float32)),
    mesh=_sc_mesh,
    scratch_types=[
        pltpu.VMEM((_CCH, D // 2), jnp.float32),
        pltpu.VMEM((_CCH, D // 2), jnp.float32),
        pltpu.VMEM((_CCH,), jnp.int32),
        pltpu.VMEM((_CCH,), jnp.int32),
        pltpu.SemaphoreType.DMA,
        pltpu.SemaphoreType.DMA,
    ],
)
def _gather2(y_hbm, pos0_hbm, pos1_hbm, y0_hbm, y1_hbm,
             a_v, b_v, i0_v, i1_v, s0, s1):
    wid = lax.axis_index("s") * _NC + lax.axis_index("c")
    for c in range(_TPW // _CCH):
        base = wid * _TPW + c * _CCH
        pltpu.sync_copy(pos0_hbm.at[pl.ds(base, _CCH)], i0_v)
        pltpu.sync_copy(pos1_hbm.at[pl.ds(base, _CCH)], i1_v)
        ga = pltpu.async_copy(y_hbm.at[i0_v], a_v, s0)
        gb = pltpu.async_copy(y_hbm.at[i1_v], b_v, s1)
        ga.wait()
        gb.wait()
        pltpu.sync_copy(a_v, y0_hbm.at[pl.ds(base, _CCH)])
        pltpu.sync_copy(b_v, y1_hbm.at[pl.ds(base, _CCH)])


def _finish_body(y0_ref, y1_ref, w_ref, o_ref):
    w0 = w_ref[0, :, 0:1]
    w1 = w_ref[1, :, 0:1]
    y0lo, y0hi = _unpack_bf16(y0_ref[...])
    y1lo, y1hi = _unpack_bf16(y1_ref[...])
    o_ref[:, :D // 2] = y0lo * w0 + y1lo * w1
    o_ref[:, D // 2:] = y0hi * w0 + y1hi * w1


def _finish(y0bf, y1bf, wrep):
    tb = T // 8
    return pl.pallas_call(
        _finish_body,
        grid=(8,),
        in_specs=[
            pl.BlockSpec((tb, D // 2), lambda i: (i, 0)),
            pl.BlockSpec((tb, D // 2), lambda i: (i, 0)),
            pl.BlockSpec((K, tb, 16), lambda i: (0, i, 0)),
        ],
        out_specs=pl.BlockSpec((tb, D), lambda i: (i, 0)),
        out_shape=jax.ShapeDtypeStruct((T, D), jnp.float32),
    )(y0bf, y1bf, wrep)


def kernel(x, router_w, gate_w, up_w, down_w):
    B, L, Dd = x.shape
    x2d = x.reshape(T, D)
    pos, wrep, be, xp = _route(x2d, router_w)
    xg = _dispatch(xp, pos[0], pos[1])
    yg = _gemm(xg, gate_w, up_w, down_w, be)
    y0p, y1p = _gather2(yg, pos[0], pos[1])
    return _finish(y0p, y1p, wrep).reshape(B, L, Dd)
